# baseline (speedup 1.0000x reference)
import jax
import jax.numpy as jnp
from jax import lax
from jax.experimental import pallas as pl
from jax.experimental.pallas import tpu as pltpu
from jax.experimental.pallas import tpu_sc as plsc

NUM_Q = 8
CODEBOOK_SIZE = 1024
CODE_DIM = 64
B, N = 16, 576

NC, NS, L = 2, 16, 16
NW = NC * NS
T = B * N
TPW = T // NW
CHUNK_T = 32
NCHUNK = TPW // CHUNK_T
ROWP = 65
GPC = TPW // L // NCHUNK


def _body(idx_hbm, cb_hbm, out_hbm, raw_v, idxq, acc, acc65, accT, *sems):
    gsems, osem = sems[:NCHUNK], sems[NCHUNK]
    wid = lax.axis_index("c") * NS + lax.axis_index("s")
    b = wid // 2
    n0 = (wid % 2) * TPW

    iota = lax.iota(jnp.int32, L)
    zeros = jnp.zeros((L,), jnp.float32)

    with jax.named_scope("ph_idx"):
        pltpu.sync_copy(idx_hbm.at[pl.ds(wid * TPW * NUM_Q, TPW * NUM_Q)], raw_v)

    copies = []
    with jax.named_scope("ph_main"):
        for c in range(NCHUNK):
            def deint(g, _):
                for q in range(NUM_Q):
                    v = plsc.load_gather(raw_v, [iota * NUM_Q + (g * L * NUM_Q + q)])
                    idxq[q, pl.ds(g * L, L)] = v
                return _

            def zrow(t, _):
                for r in range(CODE_DIM // L):
                    acc[t, pl.ds(r * L, L)] = zeros
                return _

            lax.fori_loop(c * GPC, (c + 1) * GPC, deint, 0, unroll=True)
            lax.fori_loop(c * CHUNK_T, (c + 1) * CHUNK_T, zrow, 0, unroll=8)
            dst = acc.at[pl.ds(c * CHUNK_T, CHUNK_T)]
            for q in range(NUM_Q):
                src = cb_hbm.at[q].at[idxq.at[q, pl.ds(c * CHUNK_T, CHUNK_T)]]
                cp = pltpu.make_async_copy(src, dst, gsems[c])
                cp.start(add=True)
                copies.append(cp)

    with jax.named_scope("ph_tpose"):
        def prow(t, _):
            for r in range(CODE_DIM // L):
                acc65[pl.ds(t * ROWP + r * L, L)] = acc[t, pl.ds(r * L, L)]
            return _

        colbase = iota * ROWP

        def tpass(g, _):
            rowb = g * L
            for d in range(CODE_DIM):
                v = plsc.load_gather(acc65, [colbase + (rowb * ROWP + d)])
                accT[d, pl.ds(rowb, L)] = v
            return _

        for c in range(NCHUNK):
            for cp in copies[c * NUM_Q : (c + 1) * NUM_Q]:
                cp.wait()
            lax.fori_loop(c * CHUNK_T, (c + 1) * CHUNK_T, prow, 0, unroll=8)
            lax.fori_loop(c * GPC, (c + 1) * GPC, tpass, 0, unroll=False)

    with jax.named_scope("ph_out"):
        pltpu.sync_copy(accT, out_hbm.at[b, :, pl.ds(n0, TPW)])


@jax.jit
def _vq_lookup(indices, codebooks):
    mesh = plsc.VectorSubcoreMesh(
        core_axis_name="c", subcore_axis_name="s", num_cores=NC, num_subcores=NS
    )
    f = pl.kernel(
        _body,
        out_type=jax.ShapeDtypeStruct((B, CODE_DIM, N), jnp.float32),
        mesh=mesh,
        compiler_params=pltpu.CompilerParams(
            use_tc_tiling_on_sc=False, needs_layout_passes=False
        ),
        scratch_types=[
            pltpu.VMEM((TPW * NUM_Q,), jnp.int32),
            pltpu.VMEM((NUM_Q, TPW), jnp.int32),
            pltpu.VMEM((TPW, CODE_DIM), jnp.float32),
            pltpu.VMEM((TPW * ROWP,), jnp.float32),
            pltpu.VMEM((CODE_DIM, TPW), jnp.float32),
        ]
        + [pltpu.SemaphoreType.DMA] * (NCHUNK + 1),
    )
    return f(indices, codebooks)


def kernel(indices, codebooks):
    if indices.dtype != jnp.int32:
        indices = indices.astype(jnp.int32)
    return _vq_lookup(indices.reshape(T * NUM_Q), codebooks)


# speedup vs baseline: 1.0029x; 1.0029x over previous
import jax
import jax.numpy as jnp
from jax import lax
from jax.experimental import pallas as pl
from jax.experimental.pallas import tpu as pltpu
from jax.experimental.pallas import tpu_sc as plsc

NUM_Q = 8
CODEBOOK_SIZE = 1024
CODE_DIM = 64
B, N = 16, 576

NC, NS, L = 2, 16, 16
NW = NC * NS
T = B * N
TPW = T // NW
CHUNK_T = 48
NCHUNK = TPW // CHUNK_T
ROWP = 65
GPC = TPW // L // NCHUNK


def _body(idx_hbm, cb_hbm, out_hbm, raw_v, idxq, acc, acc65, accT, *sems):
    gsems, osem = sems[:NCHUNK], sems[NCHUNK]
    wid = lax.axis_index("c") * NS + lax.axis_index("s")
    b = wid // 2
    n0 = (wid % 2) * TPW

    iota = lax.iota(jnp.int32, L)
    zeros = jnp.zeros((L,), jnp.float32)

    with jax.named_scope("ph_idx"):
        pltpu.sync_copy(idx_hbm.at[pl.ds(wid * TPW * NUM_Q, TPW * NUM_Q)], raw_v)

    copies = []
    with jax.named_scope("ph_main"):
        for c in range(NCHUNK):
            def deint(g, _):
                for q in range(NUM_Q):
                    v = plsc.load_gather(raw_v, [iota * NUM_Q + (g * L * NUM_Q + q)])
                    idxq[q, pl.ds(g * L, L)] = v
                return _

            def zrow(t, _):
                for r in range(CODE_DIM // L):
                    acc[t, pl.ds(r * L, L)] = zeros
                return _

            lax.fori_loop(c * GPC, (c + 1) * GPC, deint, 0, unroll=True)
            lax.fori_loop(c * CHUNK_T, (c + 1) * CHUNK_T, zrow, 0, unroll=8)
            dst = acc.at[pl.ds(c * CHUNK_T, CHUNK_T)]
            for q in range(NUM_Q):
                src = cb_hbm.at[q].at[idxq.at[q, pl.ds(c * CHUNK_T, CHUNK_T)]]
                cp = pltpu.make_async_copy(src, dst, gsems[c])
                cp.start(add=True)
                copies.append(cp)

    with jax.named_scope("ph_tpose"):
        def prow(t, _):
            for r in range(CODE_DIM // L):
                acc65[pl.ds(t * ROWP + r * L, L)] = acc[t, pl.ds(r * L, L)]
            return _

        colbase = iota * ROWP

        def tpass(g, _):
            rowb = g * L
            for d in range(CODE_DIM):
                v = plsc.load_gather(acc65, [colbase + (rowb * ROWP + d)])
                accT[d, pl.ds(rowb, L)] = v
            return _

        for c in range(NCHUNK):
            for cp in copies[c * NUM_Q : (c + 1) * NUM_Q]:
                cp.wait()
            lax.fori_loop(c * CHUNK_T, (c + 1) * CHUNK_T, prow, 0, unroll=8)
            lax.fori_loop(c * GPC, (c + 1) * GPC, tpass, 0, unroll=False)

    with jax.named_scope("ph_out"):
        pltpu.sync_copy(accT, out_hbm.at[b, :, pl.ds(n0, TPW)])


@jax.jit
def _vq_lookup(indices, codebooks):
    mesh = plsc.VectorSubcoreMesh(
        core_axis_name="c", subcore_axis_name="s", num_cores=NC, num_subcores=NS
    )
    f = pl.kernel(
        _body,
        out_type=jax.ShapeDtypeStruct((B, CODE_DIM, N), jnp.float32),
        mesh=mesh,
        compiler_params=pltpu.CompilerParams(
            use_tc_tiling_on_sc=False, needs_layout_passes=False
        ),
        scratch_types=[
            pltpu.VMEM((TPW * NUM_Q,), jnp.int32),
            pltpu.VMEM((NUM_Q, TPW), jnp.int32),
            pltpu.VMEM((TPW, CODE_DIM), jnp.float32),
            pltpu.VMEM((TPW * ROWP,), jnp.float32),
            pltpu.VMEM((CODE_DIM, TPW), jnp.float32),
        ]
        + [pltpu.SemaphoreType.DMA] * (NCHUNK + 1),
    )
    return f(indices, codebooks)


def kernel(indices, codebooks):
    if indices.dtype != jnp.int32:
        indices = indices.astype(jnp.int32)
    return _vq_lookup(indices.reshape(T * NUM_Q), codebooks)
